# baseline (speedup 1.0000x reference)
import functools

import jax
import jax.numpy as jnp
from jax import lax
from jax.experimental import pallas as pl
from jax.experimental.pallas import tpu as pltpu
from jax.experimental.pallas import tpu_sc as plsc

DIM = 64
BATCH = 16384
NC = 2
NS = 16
NW = NC * NS
BPW = BATCH // NW
GCHUNK = 128
NCHUNK = BPW // GCHUNK
L = 16


def _body(uid_hbm, iid_hbm, utab_hbm, itab_hbm, out_hbm,
          uidx_v, iidx_v, urows_v, irows_v, out_v, sem):
    wid = lax.axis_index("s") * NC + lax.axis_index("c")
    base = wid * BPW

    pltpu.sync_copy(uid_hbm.at[pl.ds(base, BPW)], uidx_v)
    pltpu.sync_copy(iid_hbm.at[pl.ds(base, BPW)], iidx_v)

    copies = []
    for j in range(NCHUNK):
        sl = pl.ds(j * GCHUNK, GCHUNK)
        copies.append(pltpu.async_copy(utab_hbm.at[uidx_v.at[sl]],
                                       urows_v.at[sl], sem))
        copies.append(pltpu.async_copy(itab_hbm.at[iidx_v.at[sl]],
                                       irows_v.at[sl], sem))
    for cp in copies:
        cp.wait()

    lane = lax.iota(jnp.int32, L)

    def xlane(v, t):
        return v.at[lane ^ t].get(mode="promise_in_bounds")

    def group(g, _):
        vecs = []
        for k in range(L):
            e = g * L + k
            acc = urows_v[e, pl.ds(0, L)] * irows_v[e, pl.ds(0, L)]
            for c in range(1, DIM // L):
                acc += urows_v[e, pl.ds(c * L, L)] * irows_v[e, pl.ds(c * L, L)]
            vecs.append(acc)
        t = 1
        while len(vecs) > 1:
            m = (lane & t) != 0
            vecs = [jnp.where(m, vecs[i + 1] + xlane(vecs[i + 1], t),
                              vecs[i] + xlane(vecs[i], t))
                    for i in range(0, len(vecs), 2)]
            t *= 2
        out_v[pl.ds(g * L, L)] = vecs[0]
        return _

    lax.fori_loop(0, BPW // L, group, None)

    pltpu.sync_copy(out_v, out_hbm.at[pl.ds(base, BPW)])


@jax.jit
def _scores(user_ids, item_ids, user_table, item_table):
    mesh = plsc.VectorSubcoreMesh(core_axis_name="c", subcore_axis_name="s")
    kern = functools.partial(
        pl.kernel,
        out_type=jax.ShapeDtypeStruct((BATCH,), jnp.float32),
        mesh=mesh,
        compiler_params=pltpu.CompilerParams(use_tc_tiling_on_sc=False),
        scratch_types=[
            pltpu.VMEM((BPW,), jnp.int32),
            pltpu.VMEM((BPW,), jnp.int32),
            pltpu.VMEM((BPW, DIM), jnp.float32),
            pltpu.VMEM((BPW, DIM), jnp.float32),
            pltpu.VMEM((BPW,), jnp.float32),
            pltpu.SemaphoreType.DMA,
        ],
    )(_body)
    return kern(user_ids, item_ids, user_table, item_table)


def kernel(user_ids, item_ids, user_table, item_table):
    return _scores(user_ids.astype(jnp.int32), item_ids.astype(jnp.int32),
                   user_table, item_table)
